# Initial kernel scaffold; baseline (speedup 1.0000x reference)
import jax
import jax.numpy as jnp
from jax.experimental import pallas as pl


def kernel(x_in, W, b, gamma, beta, top_n):
    raise NotImplementedError("write your pallas kernel here")


# speedup vs baseline: 1.3516x; 1.3516x over previous
import functools

import jax
import jax.numpy as jnp
from jax import lax
from jax.experimental import pallas as pl
from jax.experimental.pallas import tpu as pltpu
from jax.experimental.pallas import tpu_sc as plsc

B = 16
N = 32768
C = 128
K = 1024
TOTAL = B * N


_ROWS = 4096
_STEPS = TOTAL // _ROWS


def _score_body(x_ref, w_ref, b_ref, s_ref, stats_ref, acc_ref):
    i = pl.program_id(0)

    @pl.when(i == 0)
    def _():
        acc_ref[0] = 0.0
        acc_ref[1] = 0.0

    x_blk = x_ref[0]
    w_row = w_ref[...]
    xb = x_blk.astype(jnp.bfloat16).astype(jnp.float32)
    wb = w_row.astype(jnp.bfloat16).astype(jnp.float32)
    s_vec = jnp.sum(xb * wb, axis=1) + b_ref[0, 0]
    s_ref[0, 0, :] = s_vec
    acc_ref[0] += jnp.sum(s_vec)
    acc_ref[1] += jnp.sum(s_vec * s_vec)

    @pl.when(i == _STEPS - 1)
    def _():
        total = jnp.float32(TOTAL)
        mu = acc_ref[0] / total
        var = acc_ref[1] / total - mu * mu
        inv = 1.0 / jnp.sqrt(var + 1e-5)
        stats_ref[...] = jnp.concatenate(
            [jnp.full((1, 128), mu, jnp.float32),
             jnp.full((1, 128), inv, jnp.float32),
             jnp.zeros((6, 128), jnp.float32)], axis=0)


def _scores_and_stats(x_flat, W, b_pad):
    x3 = x_flat.reshape(_STEPS, _ROWS, C)
    s, stats = pl.pallas_call(
        _score_body,
        grid=(_STEPS,),
        in_specs=[
            pl.BlockSpec((1, _ROWS, C), lambda i: (i, 0, 0)),
            pl.BlockSpec((1, C), lambda i: (0, 0)),
            pl.BlockSpec((1, 128), lambda i: (0, 0)),
        ],
        out_specs=[
            pl.BlockSpec((1, 1, _ROWS), lambda i: (i, 0, 0)),
            pl.BlockSpec((8, 128), lambda i: (0, 0)),
        ],
        out_shape=[
            jax.ShapeDtypeStruct((_STEPS, 1, _ROWS), jnp.float32),
            jax.ShapeDtypeStruct((8, 128), jnp.float32),
        ],
        scratch_shapes=[pltpu.SMEM((2,), jnp.float32)],
    )(x3, W, b_pad)
    return s.reshape(B, N), stats


_L = 16
_NV = N // _L
_GROWS = 128
_GCHUNK = K // _GROWS
_FIELDS = ((23, 8, 0x00000000), (15, 8, 0x7F800000),
           (7, 8, 0x7FFF8000), (0, 7, 0x7FFFFF80))


def _sc_body(s_hbm, xf_hbm, mu_hbm, inv_hbm, g_hbm, be_hbm, out_hbm,
             sv, uv, hist, slots, cu, cidx, rows, vec16, sem):
    cid = lax.axis_index("c")
    sid = lax.axis_index("s")
    wid = sid * 2 + cid

    @pl.when(wid < B)
    def _():
        bidx = wid
        lanes = lax.iota(jnp.int32, _L)

        pltpu.sync_copy(mu_hbm, vec16)
        mu = vec16[...]
        pltpu.sync_copy(inv_hbm, vec16)
        inv = vec16[...]
        pltpu.sync_copy(g_hbm, vec16)
        gam = vec16[...]
        pltpu.sync_copy(be_hbm, vec16)
        bet = vec16[...]
        pltpu.sync_copy(s_hbm.at[bidx], sv)

        def key_body(i, _):
            x = sv[pl.ds(i * _L, _L)]
            v = jnp.maximum((x - mu) * inv * gam + bet, 0.0)
            u = jnp.maximum(plsc.bitcast(v, jnp.int32), 0)
            uv[pl.ds(i * _L, _L)] = u
            return 0

        lax.fori_loop(0, _NV, key_body, 0)

        ones = jnp.ones((_L,), jnp.int32)

        def select_field(carry, field):
            shift, width, himask = field
            prefix, k_rem = carry
            for j in range(_L):
                hist[pl.ds(j * _L, _L)] = jnp.zeros((_L,), jnp.int32)
            nbins = 1 << width
            prefix_v = jnp.full((_L,), prefix, jnp.int32)

            def hist_body(i, _):
                u = uv[pl.ds(i * _L, _L)]
                keep = (u & himask) == prefix_v
                binv = (u >> shift) & (nbins - 1)
                plsc.addupdate_scatter(hist, [binv], ones,
                                       mask=keep)
                return 0

            lax.fori_loop(0, _NV, hist_body, 0)

            def scan_body(j, carry):
                cum, found, binno, below = carry
                h = hist[pl.ds(j * _L, _L)]
                inc = plsc.cumsum(h)
                tot = jnp.sum(h)
                ge = (cum + inc) >= k_rem
                anyset = jnp.max(jnp.where(ge, 1, 0))
                ffs = jnp.sum(jnp.where(
                    plsc.cumsum(jnp.where(ge, 1, 0)) == 1, lanes, 0))
                excl_at = jnp.sum(jnp.where(
                    plsc.cumsum(jnp.where(ge, 1, 0)) == 1, inc - h, 0))
                first_here = (found == 0) & (anyset == 1)
                binno = jnp.where(first_here, j * _L + ffs, binno)
                below = jnp.where(first_here, cum + excl_at, below)
                found = found | anyset
                cum = cum + tot
                return cum, found, binno, below

            _, _, binno, below = lax.fori_loop(
                0, nbins // _L, scan_body,
                (jnp.int32(0), jnp.int32(0), jnp.int32(0), jnp.int32(0)))
            return (prefix | (binno << shift), k_rem - below)

        carry = (jnp.int32(0), jnp.int32(K))
        for field in _FIELDS:
            carry = select_field(carry, field)
        t, k_rem = carry
        m = K - k_rem
        t_v = jnp.full((_L,), t, jnp.int32)
        gbase = bidx * N

        def sel_body(i, carry):
            cless, ceq = carry
            u = uv[pl.ds(i * _L, _L)]
            less = u < t_v
            eq = u == t_v
            less_i = jnp.where(less, 1, 0)
            eq_i = jnp.where(eq, 1, 0)
            excl_less = plsc.cumsum(less_i) - less_i
            excl_eq = plsc.cumsum(eq_i) - eq_i
            gidx = gbase + i * _L + lanes
            rank = (m + ceq) + excl_eq
            sel = eq & (rank < K)
            hi = jnp.minimum(rank >> 7, _GCHUNK - 1)
            lo = rank & (_GROWS - 1)
            plsc.store_scatter(slots, [hi, lo], gidx, mask=sel)
            slot = cless + excl_less
            sl_clamped = jnp.minimum(slot, K - 1)
            plsc.store_scatter(cu, [sl_clamped], u, mask=less)
            plsc.store_scatter(cidx, [sl_clamped], gidx,
                               mask=less)
            return cless + jnp.sum(less_i), ceq + jnp.sum(eq_i)

        lax.fori_loop(0, _NV, sel_body, (jnp.int32(0), jnp.int32(0)))

        nbv = (m + _L - 1) // _L

        def cand_body(a, _):
            ja = a >> 4
            la = a & (_L - 1)
            va = cu[pl.ds(ja * _L, _L)]
            ia = cidx[pl.ds(ja * _L, _L)]
            ua = jnp.sum(jnp.where(lanes == la, va, 0))
            iidx = jnp.sum(jnp.where(lanes == la, ia, 0))
            ua_v = jnp.full((_L,), ua, jnp.int32)
            ii_v = jnp.full((_L,), iidx, jnp.int32)

            def rank_body(j, acc):
                ub = cu[pl.ds(j * _L, _L)]
                ib = cidx[pl.ds(j * _L, _L)]
                valid = (j * _L + lanes) < m
                lt = (ub < ua_v) | ((ub == ua_v) & (ib < ii_v))
                return acc + jnp.sum(jnp.where(valid & lt, 1, 0))

            rank = lax.fori_loop(0, nbv, rank_body, jnp.int32(0))
            hi = jnp.minimum(jnp.full((_L,), rank >> 7, jnp.int32),
                             _GCHUNK - 1)
            lo = jnp.full((_L,), rank & (_GROWS - 1), jnp.int32)
            plsc.store_scatter(slots, [hi, lo], ii_v,
                               mask=lanes == 0)
            return 0

        lax.fori_loop(0, m, cand_body, 0)

        obase = bidx * K
        for c in range(_GCHUNK):
            pltpu.async_copy(xf_hbm.at[slots.at[c]], rows, sem).wait()
            pltpu.sync_copy(rows, out_hbm.at[pl.ds(obase + c * _GROWS,
                                                   _GROWS)])


def _sc_select_gather(s, x_flat, mu16, inv16, g16, be16):
    mesh = plsc.VectorSubcoreMesh(core_axis_name="c", subcore_axis_name="s")
    kern = functools.partial(
        pl.kernel,
        out_type=jax.ShapeDtypeStruct((B * K, C), jnp.float32),
        mesh=mesh,
        scratch_types=[
            pltpu.VMEM((N,), jnp.float32),
            pltpu.VMEM((N,), jnp.int32),
            pltpu.VMEM((256,), jnp.int32),
            pltpu.VMEM((_GCHUNK, _GROWS), jnp.int32),
            pltpu.VMEM((K,), jnp.int32),
            pltpu.VMEM((K,), jnp.int32),
            pltpu.VMEM((_GROWS, C), jnp.float32),
            pltpu.VMEM((_L,), jnp.float32),
            pltpu.SemaphoreType.DMA,
        ],
        compiler_params=pltpu.CompilerParams(needs_layout_passes=False),
    )(_sc_body)
    return kern(s, x_flat, mu16, inv16, g16, be16)


def kernel(x_in, W, b, gamma, beta, top_n):
    del top_n
    x_flat = x_in.reshape(TOTAL, C)
    b_pad = jnp.broadcast_to(b.astype(jnp.float32), (1, 128))
    s, stats = _scores_and_stats(x_flat, W.astype(jnp.float32), b_pad)
    mu16 = stats[0, :_L]
    inv16 = stats[1, :_L]
    g16 = jnp.broadcast_to(gamma.astype(jnp.float32), (_L,))
    be16 = jnp.broadcast_to(beta.astype(jnp.float32), (_L,))
    out = _sc_select_gather(s, x_flat, mu16, inv16, g16, be16)
    return out.reshape(B, K, C)


# speedup vs baseline: 1.5792x; 1.1684x over previous
import functools

import jax
import jax.numpy as jnp
from jax import lax
from jax.experimental import pallas as pl
from jax.experimental.pallas import tpu as pltpu
from jax.experimental.pallas import tpu_sc as plsc

B = 16
N = 32768
C = 128
K = 1024
TOTAL = B * N


_ROWS = 4096
_STEPS = TOTAL // _ROWS


def _score_body(x_ref, w_ref, b_ref, s_ref, stats_ref, acc_ref):
    i = pl.program_id(0)

    @pl.when(i == 0)
    def _():
        acc_ref[0] = 0.0
        acc_ref[1] = 0.0

    x_blk = x_ref[0]
    w_rows = w_ref[...]
    xb = x_blk.astype(jnp.bfloat16)
    wb = w_rows.astype(jnp.bfloat16)
    s_mat = jax.lax.dot_general(
        xb, wb, (((1,), (1,)), ((), ())),
        preferred_element_type=jnp.float32)
    s_vec = jnp.sum(s_mat, axis=1) + b_ref[0, 0]
    s_ref[0, 0, :] = s_vec
    acc_ref[0] += jnp.sum(s_vec)
    acc_ref[1] += jnp.sum(s_vec * s_vec)

    @pl.when(i == _STEPS - 1)
    def _():
        total = jnp.float32(TOTAL)
        mu = acc_ref[0] / total
        var = acc_ref[1] / total - mu * mu
        inv = 1.0 / jnp.sqrt(var + 1e-5)
        stats_ref[...] = jnp.concatenate(
            [jnp.full((1, 128), mu, jnp.float32),
             jnp.full((1, 128), inv, jnp.float32),
             jnp.zeros((6, 128), jnp.float32)], axis=0)


def _scores_and_stats(x_flat, W, b_pad):
    x3 = x_flat.reshape(_STEPS, _ROWS, C)
    s, stats = pl.pallas_call(
        _score_body,
        grid=(_STEPS,),
        in_specs=[
            pl.BlockSpec((1, _ROWS, C), lambda i: (i, 0, 0)),
            pl.BlockSpec((8, C), lambda i: (0, 0)),
            pl.BlockSpec((1, 128), lambda i: (0, 0)),
        ],
        out_specs=[
            pl.BlockSpec((1, 1, _ROWS), lambda i: (i, 0, 0)),
            pl.BlockSpec((8, 128), lambda i: (0, 0)),
        ],
        out_shape=[
            jax.ShapeDtypeStruct((_STEPS, 1, _ROWS), jnp.float32),
            jax.ShapeDtypeStruct((8, 128), jnp.float32),
        ],
        scratch_shapes=[pltpu.SMEM((2,), jnp.float32)],
    )(x3, W, b_pad)
    return s.reshape(B, N), stats


_L = 16
_NV = N // _L
_GROWS = 128
_GCHUNK = K // _GROWS
_FIELDS = ((23, 8, 0x00000000), (15, 8, 0x7F800000),
           (7, 8, 0x7FFF8000), (0, 7, 0x7FFFFF80))


def _sc_body(s_hbm, xf_hbm, mu_hbm, inv_hbm, g_hbm, be_hbm, out_hbm,
             sv, uv, hist, slots, cu, cidx, rows, vec16, sem):
    cid = lax.axis_index("c")
    sid = lax.axis_index("s")
    wid = sid * 2 + cid

    @pl.when(wid < B)
    def _():
        bidx = wid
        lanes = lax.iota(jnp.int32, _L)

        pltpu.sync_copy(mu_hbm, vec16)
        mu = vec16[...]
        pltpu.sync_copy(inv_hbm, vec16)
        inv = vec16[...]
        pltpu.sync_copy(g_hbm, vec16)
        gam = vec16[...]
        pltpu.sync_copy(be_hbm, vec16)
        bet = vec16[...]
        pltpu.sync_copy(s_hbm.at[bidx], sv)

        def key_body(i, zcnt):
            x = sv[pl.ds(i * _L, _L)]
            v = jnp.maximum((x - mu) * inv * gam + bet, 0.0)
            u = jnp.maximum(plsc.bitcast(v, jnp.int32), 0)
            uv[pl.ds(i * _L, _L)] = u
            return zcnt + jnp.sum(jnp.where(u == 0, 1, 0))

        zcnt = lax.fori_loop(0, _NV, key_body, jnp.int32(0))

        ones = jnp.ones((_L,), jnp.int32)

        def select_field(carry, field):
            shift, width, himask = field
            prefix, k_rem = carry
            for j in range(_L):
                hist[pl.ds(j * _L, _L)] = jnp.zeros((_L,), jnp.int32)
            nbins = 1 << width
            prefix_v = jnp.full((_L,), prefix, jnp.int32)

            def hist_body(i, _):
                u = uv[pl.ds(i * _L, _L)]
                keep = (u & himask) == prefix_v
                binv = (u >> shift) & (nbins - 1)
                plsc.addupdate_scatter(hist, [binv], ones,
                                       mask=keep)
                return 0

            lax.fori_loop(0, _NV, hist_body, 0)

            def scan_body(j, carry):
                cum, found, binno, below = carry
                h = hist[pl.ds(j * _L, _L)]
                inc = plsc.cumsum(h)
                tot = jnp.sum(h)
                ge = (cum + inc) >= k_rem
                anyset = jnp.max(jnp.where(ge, 1, 0))
                ffs = jnp.sum(jnp.where(
                    plsc.cumsum(jnp.where(ge, 1, 0)) == 1, lanes, 0))
                excl_at = jnp.sum(jnp.where(
                    plsc.cumsum(jnp.where(ge, 1, 0)) == 1, inc - h, 0))
                first_here = (found == 0) & (anyset == 1)
                binno = jnp.where(first_here, j * _L + ffs, binno)
                below = jnp.where(first_here, cum + excl_at, below)
                found = found | anyset
                cum = cum + tot
                return cum, found, binno, below

            _, _, binno, below = lax.fori_loop(
                0, nbins // _L, scan_body,
                (jnp.int32(0), jnp.int32(0), jnp.int32(0), jnp.int32(0)))
            return (prefix | (binno << shift), k_rem - below)

        def radix_select():
            carry = (jnp.int32(0), jnp.int32(K))
            for field in _FIELDS:
                carry = select_field(carry, field)
            return carry

        t, k_rem = lax.cond(zcnt >= K,
                            lambda: (jnp.int32(0), jnp.int32(K)),
                            radix_select)
        m = K - k_rem
        t_v = jnp.full((_L,), t, jnp.int32)
        gbase = bidx * N

        def sel_body(i, carry):
            cless, ceq = carry
            u = uv[pl.ds(i * _L, _L)]
            less = u < t_v
            eq = u == t_v
            less_i = jnp.where(less, 1, 0)
            eq_i = jnp.where(eq, 1, 0)
            excl_less = plsc.cumsum(less_i) - less_i
            excl_eq = plsc.cumsum(eq_i) - eq_i
            gidx = gbase + i * _L + lanes
            rank = (m + ceq) + excl_eq
            sel = eq & (rank < K)
            hi = jnp.minimum(rank >> 7, _GCHUNK - 1)
            lo = rank & (_GROWS - 1)
            plsc.store_scatter(slots, [hi, lo], gidx, mask=sel)
            slot = cless + excl_less
            sl_clamped = jnp.minimum(slot, K - 1)
            plsc.store_scatter(cu, [sl_clamped], u, mask=less)
            plsc.store_scatter(cidx, [sl_clamped], gidx,
                               mask=less)
            return cless + jnp.sum(less_i), ceq + jnp.sum(eq_i)

        lax.fori_loop(0, _NV, sel_body, (jnp.int32(0), jnp.int32(0)))

        nbv = (m + _L - 1) // _L

        def cand_body(a, _):
            ja = a >> 4
            la = a & (_L - 1)
            va = cu[pl.ds(ja * _L, _L)]
            ia = cidx[pl.ds(ja * _L, _L)]
            ua = jnp.sum(jnp.where(lanes == la, va, 0))
            iidx = jnp.sum(jnp.where(lanes == la, ia, 0))
            ua_v = jnp.full((_L,), ua, jnp.int32)
            ii_v = jnp.full((_L,), iidx, jnp.int32)

            def rank_body(j, acc):
                ub = cu[pl.ds(j * _L, _L)]
                ib = cidx[pl.ds(j * _L, _L)]
                valid = (j * _L + lanes) < m
                lt = (ub < ua_v) | ((ub == ua_v) & (ib < ii_v))
                return acc + jnp.sum(jnp.where(valid & lt, 1, 0))

            rank = lax.fori_loop(0, nbv, rank_body, jnp.int32(0))
            hi = jnp.minimum(jnp.full((_L,), rank >> 7, jnp.int32),
                             _GCHUNK - 1)
            lo = jnp.full((_L,), rank & (_GROWS - 1), jnp.int32)
            plsc.store_scatter(slots, [hi, lo], ii_v,
                               mask=lanes == 0)
            return 0

        lax.fori_loop(0, m, cand_body, 0)

        obase = bidx * K
        for c in range(_GCHUNK):
            pltpu.async_copy(xf_hbm.at[slots.at[c]], rows, sem).wait()
            pltpu.sync_copy(rows, out_hbm.at[pl.ds(obase + c * _GROWS,
                                                   _GROWS)])


def _sc_select_gather(s, x_flat, mu16, inv16, g16, be16):
    mesh = plsc.VectorSubcoreMesh(core_axis_name="c", subcore_axis_name="s")
    kern = functools.partial(
        pl.kernel,
        out_type=jax.ShapeDtypeStruct((B * K, C), jnp.float32),
        mesh=mesh,
        scratch_types=[
            pltpu.VMEM((N,), jnp.float32),
            pltpu.VMEM((N,), jnp.int32),
            pltpu.VMEM((256,), jnp.int32),
            pltpu.VMEM((_GCHUNK, _GROWS), jnp.int32),
            pltpu.VMEM((K,), jnp.int32),
            pltpu.VMEM((K,), jnp.int32),
            pltpu.VMEM((_GROWS, C), jnp.float32),
            pltpu.VMEM((_L,), jnp.float32),
            pltpu.SemaphoreType.DMA,
        ],
        compiler_params=pltpu.CompilerParams(needs_layout_passes=False),
    )(_sc_body)
    return kern(s, x_flat, mu16, inv16, g16, be16)


def kernel(x_in, W, b, gamma, beta, top_n):
    del top_n
    x_flat = x_in.reshape(TOTAL, C)
    b_pad = jnp.broadcast_to(b.astype(jnp.float32), (1, 128))
    w_pad = jnp.concatenate(
        [W.astype(jnp.float32), jnp.zeros((7, C), jnp.float32)], axis=0)
    s, stats = _scores_and_stats(x_flat, w_pad, b_pad)
    mu16 = stats[0, :_L]
    inv16 = stats[1, :_L]
    g16 = jnp.broadcast_to(gamma.astype(jnp.float32), (_L,))
    be16 = jnp.broadcast_to(beta.astype(jnp.float32), (_L,))
    out = _sc_select_gather(s, x_flat, mu16, inv16, g16, be16)
    return out.reshape(B, K, C)


# speedup vs baseline: 2.6766x; 1.6949x over previous
import functools

import jax
import jax.numpy as jnp
from jax import lax
from jax.experimental import pallas as pl
from jax.experimental.pallas import tpu as pltpu
from jax.experimental.pallas import tpu_sc as plsc

B = 16
N = 32768
C = 128
K = 1024
TOTAL = B * N


_ROWS = 4096
_STEPS = TOTAL // _ROWS


def _score_body(x_ref, w_ref, b_ref, s_ref, stats_ref, acc_ref):
    i = pl.program_id(0)

    @pl.when(i == 0)
    def _():
        acc_ref[0] = 0.0
        acc_ref[1] = 0.0

    x_blk = x_ref[0]
    w_rows = w_ref[...]
    xb = x_blk.astype(jnp.bfloat16)
    wb = w_rows.astype(jnp.bfloat16)
    s_mat = jax.lax.dot_general(
        wb, xb, (((1,), (1,)), ((), ())),
        preferred_element_type=jnp.float32)
    s_vec = s_mat[0, :] + b_ref[0, 0]
    s_ref[0, 0, :] = s_vec
    acc_ref[0] += jnp.sum(s_vec)
    acc_ref[1] += jnp.sum(s_vec * s_vec)

    @pl.when(i == _STEPS - 1)
    def _():
        total = jnp.float32(TOTAL)
        mu = acc_ref[0] / total
        var = acc_ref[1] / total - mu * mu
        inv = 1.0 / jnp.sqrt(var + 1e-5)
        stats_ref[...] = jnp.concatenate(
            [jnp.full((1, 128), mu, jnp.float32),
             jnp.full((1, 128), inv, jnp.float32),
             jnp.zeros((6, 128), jnp.float32)], axis=0)


def _scores_and_stats(x_flat, W, b_pad):
    x3 = x_flat.reshape(_STEPS, _ROWS, C)
    s, stats = pl.pallas_call(
        _score_body,
        grid=(_STEPS,),
        in_specs=[
            pl.BlockSpec((1, _ROWS, C), lambda i: (i, 0, 0)),
            pl.BlockSpec((8, C), lambda i: (0, 0)),
            pl.BlockSpec((1, 128), lambda i: (0, 0)),
        ],
        out_specs=[
            pl.BlockSpec((1, 1, _ROWS), lambda i: (i, 0, 0)),
            pl.BlockSpec((8, 128), lambda i: (0, 0)),
        ],
        out_shape=[
            jax.ShapeDtypeStruct((_STEPS, 1, _ROWS), jnp.float32),
            jax.ShapeDtypeStruct((8, 128), jnp.float32),
        ],
        scratch_shapes=[pltpu.SMEM((2,), jnp.float32)],
    )(x3, W, b_pad)
    return s.reshape(B, N), stats


_L = 16
_NV = N // _L
_GROWS = 128
_GCHUNK = K // _GROWS
_FIELDS = ((23, 8, 0x00000000), (15, 8, 0x7F800000),
           (7, 8, 0x7FFF8000), (0, 7, 0x7FFFFF80))


def _sc_body(s_hbm, xf_hbm, mu_hbm, inv_hbm, g_hbm, be_hbm, out_hbm,
             sv, uv, hist, slots, cu, cidx, rows, vec16, sem):
    cid = lax.axis_index("c")
    sid = lax.axis_index("s")
    wid = sid * 2 + cid

    @pl.when(wid < B)
    def _():
        bidx = wid
        lanes = lax.iota(jnp.int32, _L)

        pltpu.sync_copy(mu_hbm, vec16)
        mu = vec16[...]
        pltpu.sync_copy(inv_hbm, vec16)
        inv = vec16[...]
        pltpu.sync_copy(g_hbm, vec16)
        gam = vec16[...]
        pltpu.sync_copy(be_hbm, vec16)
        bet = vec16[...]
        pltpu.sync_copy(s_hbm.at[bidx], sv)

        def key_body(i, zcnt):
            x = sv[pl.ds(i * _L, _L)]
            v = jnp.maximum((x - mu) * inv * gam + bet, 0.0)
            u = jnp.maximum(plsc.bitcast(v, jnp.int32), 0)
            uv[pl.ds(i * _L, _L)] = u
            return zcnt + jnp.sum(jnp.where(u == 0, 1, 0))

        zcnt = lax.fori_loop(0, _NV, key_body, jnp.int32(0))

        ones = jnp.ones((_L,), jnp.int32)

        def select_field(carry, field):
            shift, width, himask = field
            prefix, k_rem = carry
            for j in range(_L):
                hist[pl.ds(j * _L, _L)] = jnp.zeros((_L,), jnp.int32)
            nbins = 1 << width
            prefix_v = jnp.full((_L,), prefix, jnp.int32)

            def hist_body(i, _):
                u = uv[pl.ds(i * _L, _L)]
                keep = (u & himask) == prefix_v
                binv = (u >> shift) & (nbins - 1)
                plsc.addupdate_scatter(hist, [binv], ones,
                                       mask=keep)
                return 0

            lax.fori_loop(0, _NV, hist_body, 0)

            def scan_body(j, carry):
                cum, found, binno, below = carry
                h = hist[pl.ds(j * _L, _L)]
                inc = plsc.cumsum(h)
                tot = jnp.sum(h)
                ge = (cum + inc) >= k_rem
                anyset = jnp.max(jnp.where(ge, 1, 0))
                ffs = jnp.sum(jnp.where(
                    plsc.cumsum(jnp.where(ge, 1, 0)) == 1, lanes, 0))
                excl_at = jnp.sum(jnp.where(
                    plsc.cumsum(jnp.where(ge, 1, 0)) == 1, inc - h, 0))
                first_here = (found == 0) & (anyset == 1)
                binno = jnp.where(first_here, j * _L + ffs, binno)
                below = jnp.where(first_here, cum + excl_at, below)
                found = found | anyset
                cum = cum + tot
                return cum, found, binno, below

            _, _, binno, below = lax.fori_loop(
                0, nbins // _L, scan_body,
                (jnp.int32(0), jnp.int32(0), jnp.int32(0), jnp.int32(0)))
            return (prefix | (binno << shift), k_rem - below)

        def radix_select():
            carry = (jnp.int32(0), jnp.int32(K))
            for field in _FIELDS:
                carry = select_field(carry, field)
            return carry

        t, k_rem = lax.cond(zcnt >= K,
                            lambda: (jnp.int32(0), jnp.int32(K)),
                            radix_select)
        m = K - k_rem
        t_v = jnp.full((_L,), t, jnp.int32)
        gbase = bidx * N

        def sel_body(i, carry):
            cless, ceq = carry
            u = uv[pl.ds(i * _L, _L)]
            less = u < t_v
            eq = u == t_v
            less_i = jnp.where(less, 1, 0)
            eq_i = jnp.where(eq, 1, 0)
            excl_less = plsc.cumsum(less_i) - less_i
            excl_eq = plsc.cumsum(eq_i) - eq_i
            gidx = gbase + i * _L + lanes
            rank = (m + ceq) + excl_eq
            sel = eq & (rank < K)
            hi = jnp.minimum(rank >> 7, _GCHUNK - 1)
            lo = rank & (_GROWS - 1)
            plsc.store_scatter(slots, [hi, lo], gidx, mask=sel)
            slot = cless + excl_less
            sl_clamped = jnp.minimum(slot, K - 1)
            plsc.store_scatter(cu, [sl_clamped], u, mask=less)
            plsc.store_scatter(cidx, [sl_clamped], gidx,
                               mask=less)
            return cless + jnp.sum(less_i), ceq + jnp.sum(eq_i)

        lax.fori_loop(0, _NV, sel_body, (jnp.int32(0), jnp.int32(0)))

        nbv = (m + _L - 1) // _L

        def cand_body(a, _):
            ja = a >> 4
            la = a & (_L - 1)
            va = cu[pl.ds(ja * _L, _L)]
            ia = cidx[pl.ds(ja * _L, _L)]
            ua = jnp.sum(jnp.where(lanes == la, va, 0))
            iidx = jnp.sum(jnp.where(lanes == la, ia, 0))
            ua_v = jnp.full((_L,), ua, jnp.int32)
            ii_v = jnp.full((_L,), iidx, jnp.int32)

            def rank_body(j, acc):
                ub = cu[pl.ds(j * _L, _L)]
                ib = cidx[pl.ds(j * _L, _L)]
                valid = (j * _L + lanes) < m
                lt = (ub < ua_v) | ((ub == ua_v) & (ib < ii_v))
                return acc + jnp.sum(jnp.where(valid & lt, 1, 0))

            rank = lax.fori_loop(0, nbv, rank_body, jnp.int32(0))
            hi = jnp.minimum(jnp.full((_L,), rank >> 7, jnp.int32),
                             _GCHUNK - 1)
            lo = jnp.full((_L,), rank & (_GROWS - 1), jnp.int32)
            plsc.store_scatter(slots, [hi, lo], ii_v,
                               mask=lanes == 0)
            return 0

        lax.fori_loop(0, m, cand_body, 0)

        obase = bidx * K
        for c in range(_GCHUNK):
            pltpu.async_copy(xf_hbm.at[slots.at[c]], rows, sem).wait()
            pltpu.sync_copy(rows, out_hbm.at[pl.ds(obase + c * _GROWS,
                                                   _GROWS)])


def _sc_select_gather(s, x_flat, mu16, inv16, g16, be16):
    mesh = plsc.VectorSubcoreMesh(core_axis_name="c", subcore_axis_name="s")
    kern = functools.partial(
        pl.kernel,
        out_type=jax.ShapeDtypeStruct((B * K, C), jnp.float32),
        mesh=mesh,
        scratch_types=[
            pltpu.VMEM((N,), jnp.float32),
            pltpu.VMEM((N,), jnp.int32),
            pltpu.VMEM((256,), jnp.int32),
            pltpu.VMEM((_GCHUNK, _GROWS), jnp.int32),
            pltpu.VMEM((K,), jnp.int32),
            pltpu.VMEM((K,), jnp.int32),
            pltpu.VMEM((_GROWS, C), jnp.float32),
            pltpu.VMEM((_L,), jnp.float32),
            pltpu.SemaphoreType.DMA,
        ],
        compiler_params=pltpu.CompilerParams(needs_layout_passes=False),
    )(_sc_body)
    return kern(s, x_flat, mu16, inv16, g16, be16)


def kernel(x_in, W, b, gamma, beta, top_n):
    del top_n
    x_flat = x_in.reshape(TOTAL, C)
    b_pad = jnp.broadcast_to(b.astype(jnp.float32), (1, 128))
    w_pad = jnp.concatenate(
        [W.astype(jnp.float32), jnp.zeros((7, C), jnp.float32)], axis=0)
    s, stats = _scores_and_stats(x_flat, w_pad, b_pad)
    mu16 = stats[0, :_L]
    inv16 = stats[1, :_L]
    g16 = jnp.broadcast_to(gamma.astype(jnp.float32), (_L,))
    be16 = jnp.broadcast_to(beta.astype(jnp.float32), (_L,))
    out = _sc_select_gather(s, x_flat, mu16, inv16, g16, be16)
    return out.reshape(B, K, C)
